# Initial kernel scaffold; baseline (speedup 1.0000x reference)
import jax
import jax.numpy as jnp
from jax.experimental import pallas as pl


def kernel(x1, x2, x3, x4, x5, edge_index1, edge_index2, edge_index3, edge_index4, edge_index5, edge_attr1, edge_attr2, edge_attr3, edge_attr4, edge_attr5, W1, b1, W2, b2, linW, linb, lin1W, lin1b, lin2W, lin2b):
    raise NotImplementedError("write your pallas kernel here")


# speedup vs baseline: 154.6492x; 154.6492x over previous
import functools

import jax
import jax.numpy as jnp
from jax import lax
from jax.experimental import pallas as pl
from jax.experimental.pallas import tpu as pltpu
from jax.experimental.pallas import tpu_sc as plsc

N = 77824
E = 1245184
NG = 5
NC, NS = 2, 16
NW = NC * NS
EPW = E // NW
CK = 2048
NCHUNK = EPW // CK
RPT = N // NS
R2D = E // 128
RPW = R2D // NW
BN = 1024
F32 = jnp.float32

CKH = 1024
NPAIR = EPW // (2 * CKH)

_mesh = plsc.VectorSubcoreMesh(
    core_axis_name="c", subcore_axis_name="s", num_cores=NC, num_subcores=NS
)


def _deg_body(*refs):
    cols = refs[0:NG]
    ews = refs[NG : 2 * NG]
    zn = refs[2 * NG]
    d1o, d2o = refs[2 * NG + 1], refs[2 * NG + 2]
    colv, ewv, ew2v, d1sh, d2sh, sem = refs[2 * NG + 3 :]
    cid = lax.axis_index("c")
    sid = lax.axis_index("s")
    wid = cid * NS + sid
    r0 = sid * RPT
    for g in range(NG):
        pltpu.sync_copy(zn.at[pl.ds(r0, RPT)], d1sh.at[pl.ds(r0, RPT)])
        pltpu.sync_copy(zn.at[pl.ds(r0, RPT)], d2sh.at[pl.ds(r0, RPT)])
        plsc.subcore_barrier()
        col2d, ew2d = cols[g], ews[g]

        def chunk(c, _, col2d=col2d, ew2d=ew2d):
            rstart = wid * RPW + c * 16
            pltpu.sync_copy(col2d.at[pl.ds(rstart, 16)], colv)
            pltpu.sync_copy(ew2d.at[pl.ds(rstart, 16)], ewv)

            def jloop(j, _):
                for s in range(8):
                    v = ewv[j, pl.ds(s * 16, 16)]
                    ew2v[j, pl.ds(s * 16, 16)] = jnp.where(v > 0.5, 1.0, 0.0)
                return 0

            lax.fori_loop(0, 16, jloop, 0)
            descs = []
            for j in range(16):
                descs.append(
                    pltpu.async_copy(ewv.at[j], d1sh.at[colv.at[j]], sem, add=True)
                )
                descs.append(
                    pltpu.async_copy(ew2v.at[j], d2sh.at[colv.at[j]], sem, add=True)
                )
            for d in descs:
                d.wait()
            return 0

        lax.fori_loop(0, NCHUNK, chunk, 0)
        plsc.subcore_barrier()
        pltpu.sync_copy(d1sh.at[pl.ds(r0, RPT)], d1o.at[g, cid, pl.ds(r0, RPT)])
        pltpu.sync_copy(d2sh.at[pl.ds(r0, RPT)], d2o.at[g, cid, pl.ds(r0, RPT)])
        plsc.subcore_barrier()


_deg_call = pl.kernel(
    _deg_body,
    out_type=(
        jax.ShapeDtypeStruct((NG, NC, N), F32),
        jax.ShapeDtypeStruct((NG, NC, N), F32),
    ),
    mesh=_mesh,
    scratch_types=[
        pltpu.VMEM((16, 128), jnp.int32),
        pltpu.VMEM((16, 128), F32),
        pltpu.VMEM((16, 128), F32),
        pltpu.VMEM_SHARED((N,), F32),
        pltpu.VMEM_SHARED((N,), F32),
        pltpu.SemaphoreType.DMA,
    ],
    compiler_params=pltpu.CompilerParams(use_tc_tiling_on_sc=False),
)


CKH = 1024
NPAIR = EPW // (2 * CKH)


def _conv_body(threshold, row2d, col2d, ewflat, yref, z16, acco,
               rowv, colv, ewf, rowsv, acc_sh, semg, sems):
    cid = lax.axis_index("c")
    sid = lax.axis_index("s")
    wid = cid * NS + sid
    r0 = sid * RPT
    pltpu.sync_copy(z16.at[pl.ds(r0, RPT)], acc_sh.at[pl.ds(r0, RPT)])
    plsc.subcore_barrier()

    def pair(cc, _):
        gd = {}
        for b in range(2):
            c = 2 * cc + b
            rstart = wid * RPW + c * 8
            ebase = wid * EPW + c * CKH
            pltpu.sync_copy(row2d.at[pl.ds(rstart, 8)], rowv.at[b])
            pltpu.sync_copy(col2d.at[pl.ds(rstart, 8)], colv.at[b])
            pltpu.sync_copy(ewflat.at[pl.ds(ebase, CKH)], ewf.at[b])
            gd[b] = [
                pltpu.async_copy(
                    yref.at[rowv.at[b, j]], rowsv.at[b, pl.ds(j * 128, 128)], semg
                )
                for j in range(8)
            ]
        sd = []
        for b in range(2):
            for d in gd[b]:
                d.wait()

            def scale(t, _, b=b):
                ev = ewf[b, pl.ds(t * 16, 16)]
                if threshold:
                    ev = jnp.where(ev > 0.5, 1.0, 0.0)
                base = t * 16
                for l in range(16):
                    rowsv[b, base + l] = rowsv[b, base + l] * ev[l]
                return 0

            lax.fori_loop(0, CKH // 16, scale, 0)
            sd += [
                pltpu.async_copy(
                    rowsv.at[b, pl.ds(j * 128, 128)],
                    acc_sh.at[colv.at[b, j]],
                    sems,
                    add=True,
                )
                for j in range(8)
            ]
        for d in sd:
            d.wait()
        return 0

    lax.fori_loop(0, NPAIR, pair, 0)
    plsc.subcore_barrier()
    pltpu.sync_copy(acc_sh.at[pl.ds(r0, RPT)], acco.at[cid, pl.ds(r0, RPT)])
    plsc.subcore_barrier()


def _make_conv_call(threshold):
    return pl.kernel(
        functools.partial(_conv_body, threshold),
        out_type=jax.ShapeDtypeStruct((NC, N, 16), F32),
        mesh=_mesh,
        scratch_types=[
            pltpu.VMEM((2, 8, 128), jnp.int32),
            pltpu.VMEM((2, 8, 128), jnp.int32),
            pltpu.VMEM((2, CKH), F32),
            pltpu.VMEM((2, CKH, 16), F32),
            pltpu.VMEM_SHARED((N, 16), F32),
            pltpu.SemaphoreType.DMA,
            pltpu.SemaphoreType.DMA,
        ],
        compiler_params=pltpu.CompilerParams(use_tc_tiling_on_sc=False),
    )


_conv1_call = _make_conv_call(False)
_conv2_call = _make_conv_call(True)


NP8 = N // 8
BP = 512
GRID = NP8 // BP


def _dinvx(dP_ref, g, R):
    s = dP_ref[g, 0] + dP_ref[g, 1] + 1.0
    dv = jnp.where(s > 0, lax.rsqrt(s), 0.0)
    return lax.dot_general(dv, R, (((1,), (0,)), ((), ())),
                           preferred_element_type=F32,
                           precision=lax.Precision.HIGHEST)


def _y1_body(*refs):
    xs = refs[0:NG]
    d1P = refs[NG]
    w1k = refs[NG + 1]
    R = refs[NG + 2][...]
    y1o = refs[NG + 3 :]
    w = w1k[...]
    for g in range(NG):
        xw = lax.dot_general(xs[g][...], w, (((1,), (0,)), ((), ())),
                             preferred_element_type=F32)
        y1o[g][...] = _dinvx(d1P, g, R) * xw


def _y1_call(xs, d1P, w1k, R):
    in_specs = (
        [pl.BlockSpec((BP, 256), lambda i: (i, 0)) for _ in range(NG)]
        + [pl.BlockSpec((NG, NC, BP, 8), lambda i: (0, 0, i, 0))]
        + [pl.BlockSpec((256, 128), lambda i: (0, 0))]
        + [pl.BlockSpec((8, 128), lambda i: (0, 0))]
    )
    out_specs = [pl.BlockSpec((BP, 128), lambda i: (i, 0)) for _ in range(NG)]
    out_shape = [jax.ShapeDtypeStruct((NP8, 128), F32) for _ in range(NG)]
    return pl.pallas_call(
        _y1_body, grid=(GRID,), in_specs=in_specs, out_specs=out_specs,
        out_shape=out_shape,
    )(*xs, d1P, w1k, R)


def _mid_body(accp, y1, d1Pg, d2Pg, w2k, b1t, Rr, y2o):
    R = Rr[...]
    a = accp[0] + accp[1]
    h = jnp.maximum(_dinvx1(d1Pg, R) * (a + y1[...]) + b1t[...], 0.0)
    hw = lax.dot_general(h, w2k[...], (((1,), (0,)), ((), ())),
                         preferred_element_type=F32)
    y2o[...] = _dinvx1(d2Pg, R) * hw


def _dinvx1(dP_ref, R):
    s = dP_ref[0] + dP_ref[1] + 1.0
    dv = jnp.where(s > 0, lax.rsqrt(s), 0.0)
    return lax.dot_general(dv, R, (((1,), (0,)), ((), ())),
                           preferred_element_type=F32,
                           precision=lax.Precision.HIGHEST)


def _mid_call(accp, y1p, d1Pg, d2Pg, w2k, b1t, R):
    in_specs = [
        pl.BlockSpec((NC, BP, 128), lambda i: (0, i, 0)),
        pl.BlockSpec((BP, 128), lambda i: (i, 0)),
        pl.BlockSpec((NC, BP, 8), lambda i: (0, i, 0)),
        pl.BlockSpec((NC, BP, 8), lambda i: (0, i, 0)),
        pl.BlockSpec((128, 128), lambda i: (0, 0)),
        pl.BlockSpec((1, 128), lambda i: (0, 0)),
        pl.BlockSpec((8, 128), lambda i: (0, 0)),
    ]
    return pl.pallas_call(
        _mid_body, grid=(GRID,), in_specs=in_specs,
        out_specs=pl.BlockSpec((BP, 128), lambda i: (i, 0)),
        out_shape=jax.ShapeDtypeStruct((NP8, 128), F32),
    )(accp, y1p, d1Pg, d2Pg, w2k, b1t, R)


def _fin_body(accp, y2, d2Pg, b2t, Rr, h2o):
    a = accp[0] + accp[1]
    h2o[...] = jnp.maximum(
        _dinvx1(d2Pg, Rr[...]) * (a + y2[...]) + b2t[...], 0.0)


def _fin_call(accp, y2p, d2Pg, b2t, R):
    in_specs = [
        pl.BlockSpec((NC, BP, 128), lambda i: (0, i, 0)),
        pl.BlockSpec((BP, 128), lambda i: (i, 0)),
        pl.BlockSpec((NC, BP, 8), lambda i: (0, i, 0)),
        pl.BlockSpec((1, 128), lambda i: (0, 0)),
        pl.BlockSpec((8, 128), lambda i: (0, 0)),
    ]
    return pl.pallas_call(
        _fin_body, grid=(GRID,), in_specs=in_specs,
        out_specs=pl.BlockSpec((BP, 128), lambda i: (i, 0)),
        out_shape=jax.ShapeDtypeStruct((NP8, 128), F32),
    )(accp, y2p, d2Pg, b2t, R)


def _mlp_body(*refs):
    zs = refs[0:NG]
    lwx = refs[NG]
    lb = refs[NG + 1]
    l1w = refs[NG + 2]
    l1b = refs[NG + 3]
    l2w = refs[NG + 4]
    l2b = refs[NG + 5]
    out = refs[NG + 6]
    a = lb[...]
    for g in range(NG):
        a = a + lax.dot_general(zs[g][...], lwx[g], (((1,), (1,)), ((), ())),
                                preferred_element_type=F32)
    a = jnp.maximum(a, 0.0)
    a = jnp.maximum(
        lax.dot_general(a, l1w[...], (((1,), (1,)), ((), ())),
                        preferred_element_type=F32) + l1b[...], 0.0)
    out[...] = lax.dot_general(
        a, l2w[...], (((1,), (1,)), ((), ())), preferred_element_type=F32
    ) + l2b[...]


def _mlp_call(zs, lwx, lb, l1w, l1b, l2w, l2b):
    M = 4096
    BM = 1024
    in_specs = (
        [pl.BlockSpec((BM, 304), lambda i: (i, 0)) for _ in range(NG)]
        + [pl.BlockSpec((NG, 128, 304), lambda i: (0, 0, 0))]
        + [pl.BlockSpec((1, 128), lambda i: (0, 0))]
        + [pl.BlockSpec((32, 128), lambda i: (0, 0))]
        + [pl.BlockSpec((1, 32), lambda i: (0, 0))]
        + [pl.BlockSpec((2, 32), lambda i: (0, 0))]
        + [pl.BlockSpec((1, 2), lambda i: (0, 0))]
    )
    return pl.pallas_call(
        _mlp_body,
        grid=(M // BM,),
        in_specs=in_specs,
        out_specs=pl.BlockSpec((BM, 2), lambda i: (i, 0)),
        out_shape=jax.ShapeDtypeStruct((M, 2), F32),
    )(*zs, lwx, lb, l1w, l1b, l2w, l2b)


def kernel(x1, x2, x3, x4, x5,
           edge_index1, edge_index2, edge_index3, edge_index4, edge_index5,
           edge_attr1, edge_attr2, edge_attr3, edge_attr4, edge_attr5,
           W1, b1, W2, b2, linW, linb, lin1W, lin1b, lin2W, lin2b):
    xs = [x1, x2, x3, x4, x5]
    eis = [edge_index1, edge_index2, edge_index3, edge_index4, edge_index5]
    eas = [edge_attr1, edge_attr2, edge_attr3, edge_attr4, edge_attr5]
    rows2d = [ei[0].reshape(R2D, 128) for ei in eis]
    cols2d = [ei[1].reshape(R2D, 128) for ei in eis]
    ew2ds = [ea.reshape(R2D, 128) for ea in eas]
    zn = jnp.zeros((N,), F32)
    z16 = jnp.zeros((N, 16), F32)

    eye8 = jnp.eye(8, dtype=F32)
    R = jnp.kron(eye8, jnp.ones((1, 16), F32))
    w1k = jnp.kron(eye8, W1.T)
    w2p = jnp.concatenate([W2, jnp.zeros((14, 16), F32)], axis=0)
    w2k = jnp.kron(eye8, w2p.T)
    b1t = jnp.tile(b1, 8).reshape(1, 128)
    b2t = jnp.tile(jnp.concatenate([b2, jnp.zeros((14,), F32)]), 8).reshape(1, 128)
    lwx = jnp.zeros((NG, 128, 19, 16), F32).at[:, :, :, :2].set(
        jnp.transpose(linW.reshape(128, NG, 19, 2), (1, 0, 2, 3))
    ).reshape(NG, 128, 304)

    d1p, d2p = _deg_call(*cols2d, *ew2ds, zn)
    d1P = d1p.reshape(NG, NC, NP8, 8)
    d2P = d2p.reshape(NG, NC, NP8, 8)

    xps = [x.reshape(NP8, 256) for x in xs]
    y1ps = _y1_call(xps, d1P, w1k, R)
    y1s = [y.reshape(N, 16) for y in y1ps]

    h2ps = []
    for g in range(NG):
        acc1 = _conv1_call(rows2d[g], cols2d[g], eas[g], y1s[g], z16)
        y2p = _mid_call(acc1.reshape(NC, NP8, 128), y1ps[g], d1P[g], d2P[g],
                        w2k, b1t, R)
        acc2 = _conv2_call(rows2d[g], cols2d[g], eas[g], y2p.reshape(N, 16), z16)
        h2ps.append(
            _fin_call(acc2.reshape(NC, NP8, 128), y2p, d2P[g], b2t, R))

    zs = [h.reshape(4096, 304) for h in h2ps]
    return _mlp_call(zs, lwx, linb.reshape(1, 128), lin1W, lin1b.reshape(1, 32),
                     lin2W, lin2b.reshape(1, 2))
